# baseline (speedup 1.0000x reference)
import functools

import jax
import jax.numpy as jnp
from jax import lax
from jax.experimental import pallas as pl
from jax.experimental.pallas import tpu as pltpu
from jax.experimental.pallas import tpu_sc as plsc

TOPK = 5
LANES = 16

_DN = lax.GatherDimensionNumbers(
    offset_dims=(), collapsed_slice_dims=(0,), start_index_map=(0,)
)


def _perm(v, idx):
    return lax.gather(
        v, idx.reshape(LANES, 1), dimension_numbers=_DN, slice_sizes=(1,),
        mode=lax.GatherScatterMode.PROMISE_IN_BOUNDS,
    )


def _bfly(v, op):
    lane = jnp.arange(LANES, dtype=jnp.int32)
    for s in (8, 4, 2, 1):
        v = op(v, _perm(v, lane ^ s))
    return v


def _lex_merge(m0, i0, m1, i1):
    take = (m1 > m0) | ((m1 == m0) & (i1 < i0))
    return jnp.where(take, m1, m0), jnp.where(take, i1, i0)


def _lex_bfly(m, mi):
    lane = jnp.arange(LANES, dtype=jnp.int32)
    for s in (8, 4, 2, 1):
        m, mi = _lex_merge(m, mi, _perm(m, lane ^ s), _perm(mi, lane ^ s))
    return m, mi


def _sc_stats_kernel(B, Q, L):
    nchunk = L // LANES
    mesh = plsc.VectorSubcoreMesh(core_axis_name="c", subcore_axis_name="s")
    info = plsc.get_sparse_core_info()
    num_cores = info.num_cores

    @functools.partial(
        pl.kernel,
        out_type=jax.ShapeDtypeStruct((B, LANES), jnp.float32),
        mesh=mesh,
        scratch_types=[
            pltpu.VMEM((Q // 2, L), jnp.float32),
            pltpu.VMEM((Q // 2, L), jnp.float32),
            pltpu.VMEM((L,), jnp.float32),
            pltpu.VMEM((L,), jnp.int32),
            pltpu.VMEM((L,), jnp.float32),
            pltpu.VMEM((L,), jnp.float32),
            pltpu.VMEM((L,), jnp.int32),
            pltpu.VMEM((LANES,), jnp.float32),
            pltpu.SemaphoreType.DMA,
            pltpu.SemaphoreType.DMA,
            pltpu.SemaphoreType.DMA,
        ],
    )
    def body(teacher_hbm, logit_hbm, mask_hbm, out_hbm,
             slab_a, slab_b, xrow, mrow, counts_a, counts_b, iotab, stat,
             sem_a, sem_b, sem_xm):
        b = lax.axis_index("s") * num_cores + lax.axis_index("c")
        h_a = pltpu.async_copy(teacher_hbm.at[b, b, pl.ds(0, Q // 2)], slab_a, sem_a)
        h_b = pltpu.async_copy(teacher_hbm.at[b, b, pl.ds(Q // 2, Q // 2)], slab_b, sem_b)
        h_x = pltpu.async_copy(logit_hbm.at[b], xrow, sem_xm)
        h_m = pltpu.async_copy(mask_hbm.at[b], mrow, sem_xm)

        lane = jnp.arange(LANES, dtype=jnp.int32)
        zeros = jnp.zeros((LANES,), jnp.float32)
        neg_inf = jnp.full((LANES,), -jnp.inf, jnp.float32)
        big_i = jnp.int32(2**30)

        for c in range(nchunk):
            counts_a[pl.ds(c * LANES, LANES)] = zeros
            counts_b[pl.ds(c * LANES, LANES)] = zeros
            iotab[pl.ds(c * LANES, LANES)] = lane + c * LANES

        def sweep(slab, q):
            t = [neg_inf] * TOPK
            ti = [jnp.full((LANES,), big_i, jnp.int32)] * TOPK
            for c in range(nchunk):
                sl = pl.ds(c * LANES, LANES)
                v = slab[q, sl]
                idx = iotab[sl]
                bs = [v > t[k] for k in range(TOPK)]
                nt = [None] * TOPK
                nti = [None] * TOPK
                nt[0] = jnp.where(bs[0], v, t[0])
                nti[0] = jnp.where(bs[0], idx, ti[0])
                for k in range(1, TOPK):
                    nt[k] = jnp.where(bs[k], jnp.where(bs[k - 1], t[k - 1], v), t[k])
                    nti[k] = jnp.where(bs[k], jnp.where(bs[k - 1], ti[k - 1], idx), ti[k])
                t, ti = nt, nti
            return tuple(t) + tuple(ti)

        def extract(counts, cand):
            t, ti = cand[:TOPK], cand[TOPK:]
            d = jnp.zeros((LANES,), jnp.int32)
            for _p in range(TOPK):
                eqs = [d == k for k in range(TOPK - 1)]
                cv = t[TOPK - 1]
                ci = ti[TOPK - 1]
                for k in range(TOPK - 2, -1, -1):
                    cv = jnp.where(eqs[k], t[k], cv)
                    ci = jnp.where(eqs[k], ti[k], ci)
                _, jvec = _lex_bfly(cv, ci)
                j = jvec[0]
                cj = (j // LANES) * LANES
                hit = lane == (j - cj)
                cc = counts[pl.ds(cj, LANES)]
                counts[pl.ds(cj, LANES)] = cc + jnp.where(hit, 1.0, 0.0)
                d = d + jnp.where(ci == jvec, 1, 0)

        def per_q(i, carry):
            ca = carry[: 2 * TOPK]
            cb = carry[2 * TOPK:]
            na = sweep(slab_a, i)
            nb = sweep(slab_b, i)
            extract(counts_a, ca)
            extract(counts_b, cb)
            return na + nb

        h_a.wait()
        first_a = sweep(slab_a, 0)
        h_b.wait()
        first = first_a + sweep(slab_b, 0)
        last = lax.fori_loop(1, Q // 2, per_q, first)
        extract(counts_a, last[: 2 * TOPK])
        extract(counts_b, last[2 * TOPK:])

        h_x.wait()
        h_m.wait()

        cm = jnp.float32(Q)
        xmv = neg_inf
        for c in range(nchunk):
            xmv = jnp.maximum(xmv, xrow[pl.ds(c * LANES, LANES)])
        m_log = _bfly(xmv, jnp.maximum)

        zv = zeros
        wv = zeros
        sv = zeros
        for c in range(nchunk):
            sl = pl.ds(c * LANES, LANES)
            xv = xrow[sl]
            cc = counts_a[sl] + counts_b[sl]
            ce = jnp.where(mrow[sl] == 0, jnp.float32(-1e9), cc) - cm
            e = jnp.exp(ce)
            zv = zv + e
            wv = wv + e * (ce - xv)
            sv = sv + jnp.exp(xv - m_log)
        z = _bfly(zv, jnp.add)
        w = _bfly(wv, jnp.add)
        s = _bfly(sv, jnp.add)

        packed = jnp.where(lane == 0, z, 0.0)
        packed = packed + jnp.where(lane == 1, w, 0.0)
        packed = packed + jnp.where(lane == 2, m_log, 0.0)
        packed = packed + jnp.where(lane == 3, s, 0.0)
        stat[...] = packed
        pltpu.sync_copy(stat, out_hbm.at[b])

    return body


def _tc_finalize(stats):
    B = stats.shape[0]

    def body(s_ref, o_ref):
        s = s_ref[...]
        z = s[:, 0:1]
        w = s[:, 1:2]
        m = s[:, 2:3]
        se = s[:, 3:4]
        per_b = w / z - jnp.log(z) + m + jnp.log(se)
        o_ref[...] = jnp.sum(per_b, axis=0, keepdims=True) / B

    return pl.pallas_call(
        body,
        out_shape=jax.ShapeDtypeStruct((1, 1), jnp.float32),
    )(stats)


@jax.jit
def kernel(encoded_embeddings, pooling_logit, attention_mask, teacher_ib_term_relevance):
    B, L = pooling_logit.shape
    Q = teacher_ib_term_relevance.shape[2]
    mask_i = attention_mask.astype(jnp.int32)
    stats = _sc_stats_kernel(B, Q, L)(
        teacher_ib_term_relevance, pooling_logit, mask_i
    )
    return _tc_finalize(stats)[0, 0]


# speedup vs baseline: 1.0258x; 1.0258x over previous
import functools

import jax
import jax.numpy as jnp
from jax import lax
from jax.experimental import pallas as pl
from jax.experimental.pallas import tpu as pltpu
from jax.experimental.pallas import tpu_sc as plsc

TOPK = 5
LANES = 16

_DN = lax.GatherDimensionNumbers(
    offset_dims=(), collapsed_slice_dims=(0,), start_index_map=(0,)
)


def _perm(v, idx):
    return lax.gather(
        v, idx.reshape(LANES, 1), dimension_numbers=_DN, slice_sizes=(1,),
        mode=lax.GatherScatterMode.PROMISE_IN_BOUNDS,
    )


def _bfly(v, op):
    lane = jnp.arange(LANES, dtype=jnp.int32)
    for s in (8, 4, 2, 1):
        v = op(v, _perm(v, lane ^ s))
    return v


def _lex_merge(m0, i0, m1, i1):
    take = (m1 > m0) | ((m1 == m0) & (i1 < i0))
    return jnp.where(take, m1, m0), jnp.where(take, i1, i0)


def _lex_bfly(m, mi):
    lane = jnp.arange(LANES, dtype=jnp.int32)
    for s in (8, 4, 2, 1):
        m, mi = _lex_merge(m, mi, _perm(m, lane ^ s), _perm(mi, lane ^ s))
    return m, mi


def _sc_stats_kernel(B, Q, L):
    nchunk = L // LANES
    mesh = plsc.VectorSubcoreMesh(core_axis_name="c", subcore_axis_name="s")
    info = plsc.get_sparse_core_info()
    num_cores = info.num_cores

    @functools.partial(
        pl.kernel,
        out_type=jax.ShapeDtypeStruct((B, LANES), jnp.float32),
        mesh=mesh,
        scratch_types=[
            pltpu.VMEM((Q // 2, L), jnp.float32),
            pltpu.VMEM((Q // 2, L), jnp.float32),
            pltpu.VMEM((L,), jnp.float32),
            pltpu.VMEM((L,), jnp.int32),
            pltpu.VMEM((L,), jnp.float32),
            pltpu.VMEM((L,), jnp.float32),
            pltpu.VMEM((L,), jnp.int32),
            pltpu.VMEM((LANES,), jnp.float32),
            pltpu.SemaphoreType.DMA,
            pltpu.SemaphoreType.DMA,
            pltpu.SemaphoreType.DMA,
        ],
    )
    def body(teacher_hbm, logit_hbm, mask_hbm, out_hbm,
             slab_a, slab_b, xrow, mrow, counts_a, counts_b, iotab, stat,
             sem_a, sem_b, sem_xm):
        b = lax.axis_index("s") * num_cores + lax.axis_index("c")
        h_a = pltpu.async_copy(teacher_hbm.at[b, b, pl.ds(0, Q // 2)], slab_a, sem_a)
        h_b = pltpu.async_copy(teacher_hbm.at[b, b, pl.ds(Q // 2, Q // 2)], slab_b, sem_b)
        h_x = pltpu.async_copy(logit_hbm.at[b], xrow, sem_xm)
        h_m = pltpu.async_copy(mask_hbm.at[b], mrow, sem_xm)

        lane = jnp.arange(LANES, dtype=jnp.int32)
        zeros = jnp.zeros((LANES,), jnp.float32)
        neg_inf = jnp.full((LANES,), -jnp.inf, jnp.float32)
        big_i = jnp.int32(2**30)

        for c in range(nchunk):
            counts_a[pl.ds(c * LANES, LANES)] = zeros
            counts_b[pl.ds(c * LANES, LANES)] = zeros
            iotab[pl.ds(c * LANES, LANES)] = lane + c * LANES

        def sweep(slab, q):
            t = [neg_inf] * TOPK
            ti = [jnp.full((LANES,), big_i, jnp.int32)] * TOPK
            for c in range(nchunk):
                sl = pl.ds(c * LANES, LANES)
                v = slab[q, sl]
                idx = iotab[sl]
                bs = [v > t[k] for k in range(TOPK)]
                nt = [None] * TOPK
                nti = [None] * TOPK
                nt[0] = jnp.where(bs[0], v, t[0])
                nti[0] = jnp.where(bs[0], idx, ti[0])
                for k in range(1, TOPK):
                    nt[k] = jnp.where(bs[k], jnp.where(bs[k - 1], t[k - 1], v), t[k])
                    nti[k] = jnp.where(bs[k], jnp.where(bs[k - 1], ti[k - 1], idx), ti[k])
                t, ti = nt, nti
            return tuple(t) + tuple(ti)

        def extract(counts, cand):
            t, ti = cand[:TOPK], cand[TOPK:]
            d = jnp.zeros((LANES,), jnp.int32)
            for _p in range(TOPK):
                eqs = [d == k for k in range(TOPK - 1)]
                cv = t[TOPK - 1]
                ci = ti[TOPK - 1]
                for k in range(TOPK - 2, -1, -1):
                    cv = jnp.where(eqs[k], t[k], cv)
                    ci = jnp.where(eqs[k], ti[k], ci)
                _, jvec = _lex_bfly(cv, ci)
                j = jvec[0]
                cj = (j // LANES) * LANES
                hit = lane == (j - cj)
                cc = counts[pl.ds(cj, LANES)]
                counts[pl.ds(cj, LANES)] = cc + jnp.where(hit, 1.0, 0.0)
                d = d + jnp.where(ci == jvec, 1, 0)

        def per_q(i, carry):
            ca = carry[: 2 * TOPK]
            cb = carry[2 * TOPK:]
            na = sweep(slab_a, i)
            nb = sweep(slab_b, i)
            extract(counts_a, ca)
            extract(counts_b, cb)
            return na + nb

        h_a.wait()
        first_a = sweep(slab_a, 0)
        h_b.wait()
        first = first_a + sweep(slab_b, 0)
        last = lax.fori_loop(1, Q // 2, per_q, first)
        extract(counts_a, last[: 2 * TOPK])
        extract(counts_b, last[2 * TOPK:])

        h_x.wait()
        h_m.wait()

        cm = jnp.float32(Q)

        def red_xmax(c, xmv):
            return jnp.maximum(xmv, xrow[pl.ds(c * LANES, LANES)])

        m_log = _bfly(lax.fori_loop(0, nchunk, red_xmax, neg_inf), jnp.maximum)

        def red_sum(c, carry):
            zv, wv, sv = carry
            sl = pl.ds(c * LANES, LANES)
            xv = xrow[sl]
            cc = counts_a[sl] + counts_b[sl]
            ce = jnp.where(mrow[sl] == 0, jnp.float32(-1e9), cc) - cm
            e = jnp.exp(ce)
            return (zv + e, wv + e * (ce - xv), sv + jnp.exp(xv - m_log))

        zv, wv, sv = lax.fori_loop(0, nchunk, red_sum, (zeros, zeros, zeros))
        z = _bfly(zv, jnp.add)
        w = _bfly(wv, jnp.add)
        s = _bfly(sv, jnp.add)

        packed = jnp.where(lane == 0, z, 0.0)
        packed = packed + jnp.where(lane == 1, w, 0.0)
        packed = packed + jnp.where(lane == 2, m_log, 0.0)
        packed = packed + jnp.where(lane == 3, s, 0.0)
        stat[...] = packed
        pltpu.sync_copy(stat, out_hbm.at[b])

    return body


def _tc_finalize(stats):
    B = stats.shape[0]

    def body(s_ref, o_ref):
        s = s_ref[...]
        z = s[:, 0:1]
        w = s[:, 1:2]
        m = s[:, 2:3]
        se = s[:, 3:4]
        per_b = w / z - jnp.log(z) + m + jnp.log(se)
        o_ref[...] = jnp.sum(per_b, axis=0, keepdims=True) / B

    return pl.pallas_call(
        body,
        out_shape=jax.ShapeDtypeStruct((1, 1), jnp.float32),
    )(stats)


@jax.jit
def kernel(encoded_embeddings, pooling_logit, attention_mask, teacher_ib_term_relevance):
    B, L = pooling_logit.shape
    Q = teacher_ib_term_relevance.shape[2]
    mask_i = attention_mask.astype(jnp.int32)
    stats = _sc_stats_kernel(B, Q, L)(
        teacher_ib_term_relevance, pooling_logit, mask_i
    )
    return _tc_finalize(stats)[0, 0]
